# baseline (speedup 1.0000x reference)
import jax
import jax.numpy as jnp
from jax.experimental import pallas as pl
from jax.experimental.pallas import tpu as pltpu

N = 5000
S = 16
H = 64
GD = 4 * H
B = 200
GRID = N // B


def _lstm_block_kernel(x_ref, wih0_ref, whh0_ref, b0_ref,
                       wih1_ref, whh1_ref, b1_ref,
                       hw_ref, hb_ref, out_ref, hs_ref):
    x = x_ref[...].reshape(B * S, H)
    g0 = (jnp.dot(x, wih0_ref[...], preferred_element_type=jnp.float32)
          + b0_ref[...]).reshape(B, S, GD)

    whh0 = whh0_ref[...]
    h = jnp.zeros((B, H), jnp.float32)
    c = jnp.zeros((B, H), jnp.float32)
    for t in range(S):
        gates = g0[:, t, :] + jnp.dot(h, whh0,
                                      preferred_element_type=jnp.float32)
        i = jax.nn.sigmoid(gates[:, 0:H])
        f = jax.nn.sigmoid(gates[:, H:2 * H])
        g = jnp.tanh(gates[:, 2 * H:3 * H])
        o = jax.nn.sigmoid(gates[:, 3 * H:4 * H])
        c = f * c + i * g
        h = o * jnp.tanh(c)
        hs_ref[:, t, :] = h

    x1 = hs_ref[...].reshape(B * S, H)
    g1 = (jnp.dot(x1, wih1_ref[...], preferred_element_type=jnp.float32)
          + b1_ref[...]).reshape(B, S, GD)

    whh1 = whh1_ref[...]
    h = jnp.zeros((B, H), jnp.float32)
    c = jnp.zeros((B, H), jnp.float32)
    for t in range(S):
        gates = g1[:, t, :] + jnp.dot(h, whh1,
                                      preferred_element_type=jnp.float32)
        i = jax.nn.sigmoid(gates[:, 0:H])
        f = jax.nn.sigmoid(gates[:, H:2 * H])
        g = jnp.tanh(gates[:, 2 * H:3 * H])
        o = jax.nn.sigmoid(gates[:, 3 * H:4 * H])
        c = f * c + i * g
        h = o * jnp.tanh(c)

    out_ref[...] = (jnp.dot(h, hw_ref[...],
                            preferred_element_type=jnp.float32)
                    + hb_ref[...])


def kernel(data, label, proposals, classes,
           w_ih_0, w_hh_0, b_ih_0, b_hh_0,
           w_ih_1, w_hh_1, b_ih_1, b_hh_1,
           cls_w, cls_b, bbox_w, bbox_b):
    f32 = jnp.float32
    wih0T = w_ih_0.T
    whh0T = w_hh_0.T
    b0 = (b_ih_0 + b_hh_0).reshape(1, GD)
    wih1T = w_ih_1.T
    whh1T = w_hh_1.T
    b1 = (b_ih_1 + b_hh_1).reshape(1, GD)
    hw = jnp.concatenate([cls_w, bbox_w, jnp.zeros((1, H), f32)], axis=0).T
    hb = jnp.concatenate([cls_b, bbox_b, jnp.zeros((1,), f32)]).reshape(1, 8)

    out = pl.pallas_call(
        _lstm_block_kernel,
        grid=(GRID,),
        in_specs=[
            pl.BlockSpec((B, S, H), lambda i: (i, 0, 0)),
            pl.BlockSpec((H, GD), lambda i: (0, 0)),
            pl.BlockSpec((H, GD), lambda i: (0, 0)),
            pl.BlockSpec((1, GD), lambda i: (0, 0)),
            pl.BlockSpec((H, GD), lambda i: (0, 0)),
            pl.BlockSpec((H, GD), lambda i: (0, 0)),
            pl.BlockSpec((1, GD), lambda i: (0, 0)),
            pl.BlockSpec((H, 8), lambda i: (0, 0)),
            pl.BlockSpec((1, 8), lambda i: (0, 0)),
        ],
        out_specs=pl.BlockSpec((B, 8), lambda i: (i, 0)),
        out_shape=jax.ShapeDtypeStruct((N, 8), f32),
        scratch_shapes=[pltpu.VMEM((B, S, H), f32)],
    )(proposals, wih0T, whh0T, b0, wih1T, whh1T, b1, hw, hb)

    cls_feat = out[:, :5]
    bbox_feat = out[:, 5:7]
    return (cls_feat, bbox_feat, jnp.float32(0.0), jnp.float32(0.0))
